# Initial kernel scaffold; baseline (speedup 1.0000x reference)
import jax
import jax.numpy as jnp
from jax.experimental import pallas as pl


def kernel(x, h_x, a_x, cluster_centers):
    raise NotImplementedError("write your pallas kernel here")


# speedup vs baseline: 2.8740x; 2.8740x over previous
import jax
import jax.numpy as jnp
from jax.experimental import pallas as pl

B = 8192
D = 768
L = 256
K = 8192

BB = 512
KC = 2048


def _loss_body(x_ref, a_ref, h_ref, cc_ref, out_ref):
    i = pl.program_id(0)

    diff = x_ref[...] - a_ref[...]
    recon = jnp.sum(diff * diff)

    h = h_ref[...]
    h2 = jnp.sum(h * h, axis=1)
    hb = h.astype(jnp.bfloat16)

    def body(k, m):
        cf = cc_ref[pl.ds(k * KC, KC), :]
        c2 = jnp.sum(cf * cf, axis=1)
        s = jax.lax.dot_general(
            hb, cf.astype(jnp.bfloat16),
            (((1,), (1,)), ((), ())),
            preferred_element_type=jnp.float32,
        )
        s = s - 0.5 * c2[None, :]
        return jnp.maximum(m, jnp.max(s, axis=1))

    m = jax.lax.fori_loop(0, K // KC, body,
                          jnp.full((BB,), -jnp.inf, dtype=jnp.float32))
    d2 = h2 - 2.0 * m
    part = jnp.reshape(recon / (B * D) + jnp.sum(d2) / (B * L), (1, 1))

    @pl.when(i == 0)
    def _():
        out_ref[...] = jnp.zeros((1, 1), jnp.float32)
    out_ref[...] += part


def kernel(x, h_x, a_x, cluster_centers):
    out = pl.pallas_call(
        _loss_body,
        grid=(B // BB,),
        in_specs=[
            pl.BlockSpec((BB, D), lambda i: (i, 0)),
            pl.BlockSpec((BB, D), lambda i: (i, 0)),
            pl.BlockSpec((BB, L), lambda i: (i, 0)),
            pl.BlockSpec((K, L), lambda i: (0, 0)),
        ],
        out_specs=pl.BlockSpec((1, 1), lambda i: (0, 0)),
        out_shape=jax.ShapeDtypeStruct((1, 1), jnp.float32),
    )(x, a_x, h_x, cluster_centers)
    return out[0, 0]


# speedup vs baseline: 4.3805x; 1.5242x over previous
import jax
import jax.numpy as jnp
from jax.experimental import pallas as pl
from jax.experimental.pallas import tpu as pltpu

B = 8192
D = 768
L = 256
K = 8192

BB = 512
LANES = 128


def _loss_body(x_ref, a_ref, h_ref, cc_ref, out_ref, c2_ref):
    i = pl.program_id(0)

    @pl.when(i == 0)
    def _():
        cf = cc_ref[...]
        c2 = jnp.sum(cf * cf, axis=1)
        c2_ref[...] = (0.5 * c2).reshape(1, K)

    diff = x_ref[...] - a_ref[...]
    recon = jnp.sum(diff * diff)

    h = h_ref[...]
    h2 = jnp.sum(h * h, axis=1)

    s = jax.lax.dot_general(
        h.astype(jnp.bfloat16), cc_ref[...].astype(jnp.bfloat16),
        (((1,), (1,)), ((), ())),
        preferred_element_type=jnp.float32,
    )

    m = jnp.full((BB, LANES), -jnp.inf, dtype=jnp.float32)
    for t in range(K // LANES):
        sl = slice(t * LANES, (t + 1) * LANES)
        m = jnp.maximum(m, s[:, sl] - c2_ref[0:1, sl])
    m_row = jnp.max(m, axis=1)

    d2 = h2 - 2.0 * m_row
    part = jnp.reshape(recon / (B * D) + jnp.sum(d2) / (B * L), (1, 1))

    @pl.when(i == 0)
    def _():
        out_ref[...] = jnp.zeros((1, 1), jnp.float32)
    out_ref[...] += part


def kernel(x, h_x, a_x, cluster_centers):
    out = pl.pallas_call(
        _loss_body,
        grid=(B // BB,),
        in_specs=[
            pl.BlockSpec((BB, D), lambda i: (i, 0)),
            pl.BlockSpec((BB, D), lambda i: (i, 0)),
            pl.BlockSpec((BB, L), lambda i: (i, 0)),
            pl.BlockSpec((K, L), lambda i: (0, 0)),
        ],
        out_specs=pl.BlockSpec((1, 1), lambda i: (0, 0)),
        out_shape=jax.ShapeDtypeStruct((1, 1), jnp.float32),
        scratch_shapes=[pltpu.VMEM((1, K), jnp.float32)],
    )(x, a_x, h_x, cluster_centers)
    return out[0, 0]


# speedup vs baseline: 4.4318x; 1.0117x over previous
import jax
import jax.numpy as jnp
from jax.experimental import pallas as pl
from jax.experimental.pallas import tpu as pltpu

B = 8192
D = 768
L = 256
K = 8192

BB = 512
LANES = 128


def _loss_body(x_ref, a_ref, h_ref, cc_ref, out_ref, c2_ref, ccb_ref):
    i = pl.program_id(0)

    @pl.when(i == 0)
    def _():
        cf = cc_ref[...]
        c2 = jnp.sum(cf * cf, axis=1)
        c2_ref[...] = (0.5 * c2).reshape(1, K)
        ccb_ref[...] = cf.astype(jnp.bfloat16)

    diff = x_ref[...] - a_ref[...]
    recon = jnp.sum(diff * diff)

    h = h_ref[...]
    h2 = jnp.sum(h * h, axis=1)

    s = jax.lax.dot_general(
        h.astype(jnp.bfloat16), ccb_ref[...],
        (((1,), (1,)), ((), ())),
        preferred_element_type=jnp.float32,
    )

    m = jnp.full((BB, LANES), -jnp.inf, dtype=jnp.float32)
    for t in range(K // LANES):
        sl = slice(t * LANES, (t + 1) * LANES)
        m = jnp.maximum(m, s[:, sl] - c2_ref[0:1, sl])
    m_row = jnp.max(m, axis=1)

    d2 = h2 - 2.0 * m_row
    part = jnp.reshape(recon / (B * D) + jnp.sum(d2) / (B * L), (1, 1))

    @pl.when(i == 0)
    def _():
        out_ref[...] = jnp.zeros((1, 1), jnp.float32)
    out_ref[...] += part


def kernel(x, h_x, a_x, cluster_centers):
    out = pl.pallas_call(
        _loss_body,
        grid=(B // BB,),
        in_specs=[
            pl.BlockSpec((BB, D), lambda i: (i, 0)),
            pl.BlockSpec((BB, D), lambda i: (i, 0)),
            pl.BlockSpec((BB, L), lambda i: (i, 0)),
            pl.BlockSpec((K, L), lambda i: (0, 0)),
        ],
        out_specs=pl.BlockSpec((1, 1), lambda i: (0, 0)),
        out_shape=jax.ShapeDtypeStruct((1, 1), jnp.float32),
        scratch_shapes=[pltpu.VMEM((1, K), jnp.float32),
                        pltpu.VMEM((K, L), jnp.bfloat16)],
    )(x, a_x, h_x, cluster_centers)
    return out[0, 0]
